# baseline (speedup 1.0000x reference)
import functools

import jax
import jax.numpy as jnp
from jax import lax
from jax.experimental import pallas as pl
from jax.experimental.pallas import tpu as pltpu
from jax.experimental.pallas import tpu_sc as plsc

_N = 10000
_E = 320000
_NREL = 2

_NC = 2
_NS = 16
_NW = _NC * _NS
_CHUNK = 80
_EW = _E // _NW
_NCHUNK = _EW // _CHUNK
_RPT = 624
_TAIL0 = _NS * _RPT
_TAIL = _N - _TAIL0


_NB = 4
_NBI = 8
_ILEAD = 6
_GLEAD = _NB - 1


@functools.lru_cache(maxsize=None)
def _make_sc_agg(d):
    mesh = plsc.VectorSubcoreMesh(core_axis_name="c", subcore_axis_name="s")

    @functools.partial(
        pl.kernel,
        mesh=mesh,
        compiler_params=pltpu.CompilerParams(use_tc_tiling_on_sc=(d % 128 == 0)),
        out_type=jax.ShapeDtypeStruct((_NC, _N, d), jnp.float32),
        scratch_types=[
            [pltpu.VMEM((_CHUNK,), jnp.int32) for _ in range(_NBI)],
            [pltpu.VMEM((_CHUNK,), jnp.int32) for _ in range(_NBI)],
            [pltpu.VMEM((_CHUNK,), jnp.int32) for _ in range(_NBI)],
            [pltpu.VMEM((_CHUNK, d), jnp.float32) for _ in range(_NB)],
            [pltpu.SemaphoreType.DMA for _ in range(_NBI)],
            [pltpu.SemaphoreType.DMA for _ in range(_NB)],
            [pltpu.SemaphoreType.DMA for _ in range(_NB)],
            pltpu.VMEM_SHARED((_N, d), jnp.float32),
        ],
    )
    def sc_agg(table_hbm, src_hbm, dst_hbm, et_hbm, out_hbm,
               idsb, etb, dstb, rows, sem_i, sem_g, sem_s, acc_sh):
        cid = lax.axis_index("c")
        sid = lax.axis_index("s")
        base = cid * (_E // _NC) + sid * _EW
        row0 = sid * _RPT

        zvec = jnp.zeros((16,), jnp.float32)

        @pl.loop(0, _CHUNK)
        def _zero_row(r):
            for c in range(d // 16):
                rows[0][r, pl.ds(c * 16, 16)] = zvec

        for t in range(_RPT // _CHUNK):
            pltpu.sync_copy(rows[0], acc_sh.at[pl.ds(row0 + t * _CHUNK, _CHUNK)])
        rem = _RPT % _CHUNK
        if rem:
            pltpu.sync_copy(rows[0].at[pl.ds(0, rem)],
                            acc_sh.at[pl.ds(row0 + (_RPT // _CHUNK) * _CHUNK, rem)])

        @pl.when(sid == _NS - 1)
        def _zero_tail():
            pltpu.sync_copy(rows[0].at[pl.ds(0, _TAIL)],
                            acc_sh.at[pl.ds(_TAIL0, _TAIL)])

        plsc.subcore_barrier()

        def fire_idx(j, bi):
            off = base + j * _CHUNK
            pltpu.async_copy(src_hbm.at[pl.ds(off, _CHUNK)], idsb[bi], sem_i[bi])
            pltpu.async_copy(et_hbm.at[pl.ds(off, _CHUNK)], etb[bi], sem_i[bi])
            pltpu.async_copy(dst_hbm.at[pl.ds(off, _CHUNK)], dstb[bi], sem_i[bi])

        def ready_gather(jf, bi, b4):
            pltpu.make_async_copy(src_hbm.at[pl.ds(base, _CHUNK)],
                                  idsb[bi], sem_i[bi]).wait()
            pltpu.make_async_copy(et_hbm.at[pl.ds(base, _CHUNK)],
                                  etb[bi], sem_i[bi]).wait()
            pltpu.make_async_copy(dst_hbm.at[pl.ds(base, _CHUNK)],
                                  dstb[bi], sem_i[bi]).wait()
            for t in range(_CHUNK // 16):
                sl = pl.ds(t * 16, 16)
                idsb[bi][sl] = etb[bi][sl] * _N + idsb[bi][sl]
            pltpu.async_copy(table_hbm.at[idsb[bi]], rows[b4], sem_g[b4])

        def wait_scatter(b4, b8):
            pltpu.make_async_copy(rows[b4], acc_sh.at[dstb[b8]],
                                  sem_s[b4]).wait()

        def step(b8, dyn_j, guard_ws=False, guard_fi=False, last=None):
            b4 = b8 % _NB
            pltpu.make_async_copy(table_hbm.at[idsb[b8]], rows[b4],
                                  sem_g[b4]).wait()
            pltpu.async_copy(rows[b4], acc_sh.at[dstb[b8]], sem_s[b4],
                             add=True)
            if guard_ws:
                @pl.when(dyn_j > 0)
                def _ws():
                    wait_scatter((b4 - 1) % _NB, (b8 - 1) % _NBI)
            else:
                wait_scatter((b4 - 1) % _NB, (b8 - 1) % _NBI)
            if last is None or last >= _GLEAD:
                ready_gather(dyn_j + _GLEAD, (b8 + _GLEAD) % _NBI,
                             (b4 + _GLEAD) % _NB)
            if last is None or last >= _ILEAD:
                if guard_fi:
                    @pl.when(dyn_j + _ILEAD < _NCHUNK)
                    def _fi():
                        fire_idx(dyn_j + _ILEAD, (b8 + _ILEAD) % _NBI)
                else:
                    fire_idx(dyn_j + _ILEAD, (b8 + _ILEAD) % _NBI)

        for c in range(_ILEAD):
            fire_idx(c, c)
        for c in range(_GLEAD):
            ready_gather(c, c, c)

        @pl.loop(0, _NCHUNK // _NBI)
        def _grp(g):
            j0 = g * _NBI
            for b8 in range(_NBI):
                step(b8, j0 + b8, guard_ws=(b8 == 0), guard_fi=(b8 == _NBI - 1))

        for j in range(_NCHUNK - (_NCHUNK % _NBI), _NCHUNK):
            step(j % _NBI, j, last=_NCHUNK - 1 - j)

        wait_scatter((_NCHUNK - 1) % _NB, (_NCHUNK - 1) % _NBI)

        plsc.subcore_barrier()
        pltpu.sync_copy(acc_sh.at[pl.ds(row0, _RPT)],
                        out_hbm.at[cid, pl.ds(row0, _RPT)])

        @pl.when(sid == _NS - 1)
        def _write_tail():
            pltpu.sync_copy(acc_sh.at[pl.ds(_TAIL0, _TAIL)],
                            out_hbm.at[cid, pl.ds(_TAIL0, _TAIL)])

    return sc_agg


_BN = 1000
_GRID = _N // _BN


def _mm(a, b):
    return lax.dot(a.astype(jnp.bfloat16), b.astype(jnp.bfloat16),
                   preferred_element_type=jnp.float32)


def _dense0_body(h_ref, w_ref, wl_ref, b_ref, table_ref, self_ref):
    h = h_ref[...]
    w = w_ref[...]
    table_ref[0] = _mm(h, w[0])
    table_ref[1] = _mm(h, w[1])
    self_ref[...] = _mm(h, wl_ref[...]) + b_ref[...]


def _densef_body(agg_ref, selfp_ref, w_ref, wl_ref, b_ref, table_ref, self_ref):
    h = jnp.maximum(agg_ref[0] + agg_ref[1] + selfp_ref[...], 0.0)
    w = w_ref[...]
    table_ref[0] = _mm(h, w[0])
    table_ref[1] = _mm(h, w[1])
    self_ref[...] = _mm(h, wl_ref[...]) + b_ref[...]


def _dense_outs(dout):
    return (
        [
            pl.BlockSpec((_NREL, _BN, dout), lambda i: (0, i, 0)),
            pl.BlockSpec((_BN, dout), lambda i: (i, 0)),
        ],
        [
            jax.ShapeDtypeStruct((_NREL, _N, dout), jnp.float32),
            jax.ShapeDtypeStruct((_N, dout), jnp.float32),
        ],
    )


def _dense0(h, W, Wl, b):
    din, dout = Wl.shape
    out_specs, out_shape = _dense_outs(dout)
    return pl.pallas_call(
        _dense0_body,
        grid=(_GRID,),
        in_specs=[
            pl.BlockSpec((_BN, din), lambda i: (i, 0)),
            pl.BlockSpec((_NREL, din, dout), lambda i: (0, 0, 0)),
            pl.BlockSpec((din, dout), lambda i: (0, 0)),
            pl.BlockSpec((1, dout), lambda i: (0, 0)),
        ],
        out_specs=out_specs,
        out_shape=out_shape,
    )(h, W, Wl, b.reshape(1, dout))


def _densef(agg, selfp, W, Wl, b):
    din, dout = Wl.shape
    out_specs, out_shape = _dense_outs(dout)
    return pl.pallas_call(
        _densef_body,
        grid=(_GRID,),
        in_specs=[
            pl.BlockSpec((_NC, _BN, din), lambda i: (0, i, 0)),
            pl.BlockSpec((_BN, din), lambda i: (i, 0)),
            pl.BlockSpec((_NREL, din, dout), lambda i: (0, 0, 0)),
            pl.BlockSpec((din, dout), lambda i: (0, 0)),
            pl.BlockSpec((1, dout), lambda i: (0, 0)),
        ],
        out_specs=out_specs,
        out_shape=out_shape,
    )(agg, selfp, W, Wl, b.reshape(1, dout))


def _combine_body(agg_ref, selfp_ref, out_ref):
    out_ref[...] = agg_ref[0] + agg_ref[1] + selfp_ref[...]


def _combine(agg, selfp):
    dout = selfp.shape[1]
    return pl.pallas_call(
        _combine_body,
        grid=(_GRID,),
        in_specs=[
            pl.BlockSpec((_NC, _BN, dout), lambda i: (0, i, 0)),
            pl.BlockSpec((_BN, dout), lambda i: (i, 0)),
        ],
        out_specs=pl.BlockSpec((_BN, dout), lambda i: (i, 0)),
        out_shape=jax.ShapeDtypeStruct((_N, dout), jnp.float32),
    )(agg, selfp)


def kernel(features, edge_index, edge_type, W0, Wl0, b0, W1, Wl1, b1, W2, Wl2, b2):
    sc128 = _make_sc_agg(128)
    sc64 = _make_sc_agg(64)
    src = edge_index[0]
    dst = edge_index[1]
    table0, self0 = _dense0(features, W0, Wl0, b0)
    agg0 = sc128(table0.reshape(_NREL * _N, 128), src, dst, edge_type)
    table1, self1 = _densef(agg0, self0, W1, Wl1, b1)
    agg1 = sc128(table1.reshape(_NREL * _N, 128), src, dst, edge_type)
    table2, self2 = _densef(agg1, self1, W2, Wl2, b2)
    agg2 = sc64(table2.reshape(_NREL * _N, 64), src, dst, edge_type)
    return _combine(agg2, self2)


# speedup vs baseline: 1.0060x; 1.0060x over previous
import functools

import jax
import jax.numpy as jnp
from jax import lax
from jax.experimental import pallas as pl
from jax.experimental.pallas import tpu as pltpu
from jax.experimental.pallas import tpu_sc as plsc

_N = 10000
_E = 320000
_NREL = 2

_NC = 2
_NS = 16
_NW = _NC * _NS
_CHUNK = 80
_EW = _E // _NW
_NCHUNK = _EW // _CHUNK
_RPT = 624
_TAIL0 = _NS * _RPT
_TAIL = _N - _TAIL0


_NB = 4
_NBI = 8
_ILEAD = 6
_GLEAD = _NB - 1


@functools.lru_cache(maxsize=None)
def _make_sc_agg(d):
    mesh = plsc.VectorSubcoreMesh(core_axis_name="c", subcore_axis_name="s")

    @functools.partial(
        pl.kernel,
        mesh=mesh,
        compiler_params=pltpu.CompilerParams(use_tc_tiling_on_sc=(d % 128 == 0)),
        out_type=jax.ShapeDtypeStruct((_NC, _N, d), jnp.float32),
        scratch_types=[
            [pltpu.VMEM((_CHUNK,), jnp.int32) for _ in range(_NBI)],
            [pltpu.VMEM((_CHUNK,), jnp.int32) for _ in range(_NBI)],
            [pltpu.VMEM((_CHUNK,), jnp.int32) for _ in range(_NBI)],
            [pltpu.VMEM((_CHUNK, d), jnp.float32) for _ in range(_NB)],
            [pltpu.SemaphoreType.DMA for _ in range(_NBI)],
            [pltpu.SemaphoreType.DMA for _ in range(_NB)],
            [pltpu.SemaphoreType.DMA for _ in range(_NB)],
            pltpu.VMEM_SHARED((_N, d), jnp.float32),
        ],
    )
    def sc_agg(table_hbm, src_hbm, dst_hbm, et_hbm, out_hbm,
               idsb, etb, dstb, rows, sem_i, sem_g, sem_s, acc_sh):
        cid = lax.axis_index("c")
        sid = lax.axis_index("s")
        base = cid * (_E // _NC) + sid * _EW
        row0 = sid * _RPT

        zvec = jnp.zeros((16,), jnp.float32)

        @pl.loop(0, _CHUNK)
        def _zero_row(r):
            for c in range(d // 16):
                rows[0][r, pl.ds(c * 16, 16)] = zvec

        for t in range(_RPT // _CHUNK):
            pltpu.sync_copy(rows[0], acc_sh.at[pl.ds(row0 + t * _CHUNK, _CHUNK)])
        rem = _RPT % _CHUNK
        if rem:
            pltpu.sync_copy(rows[0].at[pl.ds(0, rem)],
                            acc_sh.at[pl.ds(row0 + (_RPT // _CHUNK) * _CHUNK, rem)])

        @pl.when(sid == _NS - 1)
        def _zero_tail():
            pltpu.sync_copy(rows[0].at[pl.ds(0, _TAIL)],
                            acc_sh.at[pl.ds(_TAIL0, _TAIL)])

        plsc.subcore_barrier()

        def fire_idx(j, bi):
            off = base + j * _CHUNK
            pltpu.async_copy(src_hbm.at[pl.ds(off, _CHUNK)], idsb[bi], sem_i[bi])
            pltpu.async_copy(et_hbm.at[pl.ds(off, _CHUNK)], etb[bi], sem_i[bi])
            pltpu.async_copy(dst_hbm.at[pl.ds(off, _CHUNK)], dstb[bi], sem_i[bi])

        def ready_gather(jf, bi, b4):
            pltpu.make_async_copy(src_hbm.at[pl.ds(base, _CHUNK)],
                                  idsb[bi], sem_i[bi]).wait()
            pltpu.make_async_copy(et_hbm.at[pl.ds(base, _CHUNK)],
                                  etb[bi], sem_i[bi]).wait()
            pltpu.make_async_copy(dst_hbm.at[pl.ds(base, _CHUNK)],
                                  dstb[bi], sem_i[bi]).wait()
            for t in range(_CHUNK // 16):
                sl = pl.ds(t * 16, 16)
                idsb[bi][sl] = etb[bi][sl] * _N + idsb[bi][sl]
            pltpu.async_copy(table_hbm.at[idsb[bi]], rows[b4], sem_g[b4])

        def wait_scatter(b4, b8):
            pltpu.make_async_copy(rows[b4], acc_sh.at[dstb[b8]],
                                  sem_s[b4]).wait()

        def step(b8, dyn_j, guard_ws=False, guard_fi=False, last=None):
            b4 = b8 % _NB
            pltpu.make_async_copy(table_hbm.at[idsb[b8]], rows[b4],
                                  sem_g[b4]).wait()
            pltpu.async_copy(rows[b4], acc_sh.at[dstb[b8]], sem_s[b4],
                             add=True)
            if guard_ws:
                @pl.when(dyn_j > 0)
                def _ws():
                    wait_scatter((b4 - 1) % _NB, (b8 - 1) % _NBI)
            else:
                wait_scatter((b4 - 1) % _NB, (b8 - 1) % _NBI)
            if last is None or last >= _GLEAD:
                ready_gather(dyn_j + _GLEAD, (b8 + _GLEAD) % _NBI,
                             (b4 + _GLEAD) % _NB)
            if last is None or last >= _ILEAD:
                if guard_fi:
                    @pl.when(dyn_j + _ILEAD < _NCHUNK)
                    def _fi():
                        fire_idx(dyn_j + _ILEAD, (b8 + _ILEAD) % _NBI)
                else:
                    fire_idx(dyn_j + _ILEAD, (b8 + _ILEAD) % _NBI)

        for c in range(_ILEAD):
            fire_idx(c, c)
        for c in range(_GLEAD):
            ready_gather(c, c, c)

        @pl.loop(0, _NCHUNK // _NBI)
        def _grp(g):
            j0 = g * _NBI
            for b8 in range(_NBI):
                step(b8, j0 + b8, guard_ws=(b8 == 0), guard_fi=(b8 == _NBI - 1))

        for j in range(_NCHUNK - (_NCHUNK % _NBI), _NCHUNK):
            step(j % _NBI, j, last=_NCHUNK - 1 - j)

        wait_scatter((_NCHUNK - 1) % _NB, (_NCHUNK - 1) % _NBI)

        plsc.subcore_barrier()
        pltpu.sync_copy(acc_sh.at[pl.ds(row0, _RPT)],
                        out_hbm.at[cid, pl.ds(row0, _RPT)])

        @pl.when(sid == _NS - 1)
        def _write_tail():
            pltpu.sync_copy(acc_sh.at[pl.ds(_TAIL0, _TAIL)],
                            out_hbm.at[cid, pl.ds(_TAIL0, _TAIL)])

    return sc_agg


_BN = 1000
_GRID = _N // _BN


def _mm(a, b):
    return lax.dot(a.astype(jnp.bfloat16), b.astype(jnp.bfloat16),
                   preferred_element_type=jnp.float32)


def _table0_body(h_ref, w_ref, table_ref):
    h = h_ref[...]
    w = w_ref[...]
    table_ref[0] = _mm(h, w[0])
    table_ref[1] = _mm(h, w[1])


def _tablef_body(agg_ref, selfp_ref, w_ref, table_ref, h_ref):
    h = jnp.maximum(agg_ref[0] + agg_ref[1] + selfp_ref[...], 0.0)
    w = w_ref[...]
    table_ref[0] = _mm(h, w[0])
    table_ref[1] = _mm(h, w[1])
    h_ref[...] = h


def _self_body(h_ref, wl_ref, b_ref, self_ref):
    self_ref[...] = _mm(h_ref[...], wl_ref[...]) + b_ref[...]


def _table0(h, W):
    din, dout = W.shape[1:]
    return pl.pallas_call(
        _table0_body,
        grid=(_GRID,),
        in_specs=[
            pl.BlockSpec((_BN, din), lambda i: (i, 0)),
            pl.BlockSpec((_NREL, din, dout), lambda i: (0, 0, 0)),
        ],
        out_specs=pl.BlockSpec((_NREL, _BN, dout), lambda i: (0, i, 0)),
        out_shape=jax.ShapeDtypeStruct((_NREL, _N, dout), jnp.float32),
    )(h, W)


def _tablef(agg, selfp, W):
    din, dout = W.shape[1:]
    return pl.pallas_call(
        _tablef_body,
        grid=(_GRID,),
        in_specs=[
            pl.BlockSpec((_NC, _BN, din), lambda i: (0, i, 0)),
            pl.BlockSpec((_BN, din), lambda i: (i, 0)),
            pl.BlockSpec((_NREL, din, dout), lambda i: (0, 0, 0)),
        ],
        out_specs=[
            pl.BlockSpec((_NREL, _BN, dout), lambda i: (0, i, 0)),
            pl.BlockSpec((_BN, din), lambda i: (i, 0)),
        ],
        out_shape=[
            jax.ShapeDtypeStruct((_NREL, _N, dout), jnp.float32),
            jax.ShapeDtypeStruct((_N, din), jnp.float32),
        ],
    )(agg, selfp, W)


def _selfk(h, Wl, b):
    din, dout = Wl.shape
    return pl.pallas_call(
        _self_body,
        grid=(_GRID,),
        in_specs=[
            pl.BlockSpec((_BN, din), lambda i: (i, 0)),
            pl.BlockSpec((din, dout), lambda i: (0, 0)),
            pl.BlockSpec((1, dout), lambda i: (0, 0)),
        ],
        out_specs=pl.BlockSpec((_BN, dout), lambda i: (i, 0)),
        out_shape=jax.ShapeDtypeStruct((_N, dout), jnp.float32),
    )(h, Wl, b.reshape(1, dout))


def _combine_body(agg_ref, selfp_ref, out_ref):
    out_ref[...] = agg_ref[0] + agg_ref[1] + selfp_ref[...]


def _combine(agg, selfp):
    dout = selfp.shape[1]
    return pl.pallas_call(
        _combine_body,
        grid=(_GRID,),
        in_specs=[
            pl.BlockSpec((_NC, _BN, dout), lambda i: (0, i, 0)),
            pl.BlockSpec((_BN, dout), lambda i: (i, 0)),
        ],
        out_specs=pl.BlockSpec((_BN, dout), lambda i: (i, 0)),
        out_shape=jax.ShapeDtypeStruct((_N, dout), jnp.float32),
    )(agg, selfp)


def kernel(features, edge_index, edge_type, W0, Wl0, b0, W1, Wl1, b1, W2, Wl2, b2):
    sc128 = _make_sc_agg(128)
    sc64 = _make_sc_agg(64)
    src = edge_index[0]
    dst = edge_index[1]
    table0 = _table0(features, W0)
    agg0 = sc128(table0.reshape(_NREL * _N, 128), src, dst, edge_type)
    self0 = _selfk(features, Wl0, b0)
    table1, h1 = _tablef(agg0, self0, W1)
    agg1 = sc128(table1.reshape(_NREL * _N, 128), src, dst, edge_type)
    self1 = _selfk(h1, Wl1, b1)
    table2, h2 = _tablef(agg1, self1, W2)
    agg2 = sc64(table2.reshape(_NREL * _N, 64), src, dst, edge_type)
    self2 = _selfk(h2, Wl2, b2)
    return _combine(agg2, self2)


# speedup vs baseline: 1.0305x; 1.0244x over previous
import functools

import jax
import jax.numpy as jnp
from jax import lax
from jax.experimental import pallas as pl
from jax.experimental.pallas import tpu as pltpu
from jax.experimental.pallas import tpu_sc as plsc

_N = 10000
_E = 320000
_NREL = 2

_NC = 2
_NS = 16
_NW = _NC * _NS
_CHUNK = 80
_EW = _E // _NW
_NCHUNK = _EW // _CHUNK
_RPT = 624
_TAIL0 = _NS * _RPT
_TAIL = _N - _TAIL0


_NB = 4
_NBI = 8
_ILEAD = 6
_GLEAD = _NB - 1


@functools.lru_cache(maxsize=None)
def _make_sc_agg(d):
    mesh = plsc.VectorSubcoreMesh(core_axis_name="c", subcore_axis_name="s")

    @functools.partial(
        pl.kernel,
        mesh=mesh,
        compiler_params=pltpu.CompilerParams(use_tc_tiling_on_sc=(d % 128 == 0)),
        out_type=jax.ShapeDtypeStruct((_NC, _N, d), jnp.float32),
        scratch_types=[
            [pltpu.VMEM((_CHUNK,), jnp.int32) for _ in range(_NBI)],
            [pltpu.VMEM((_CHUNK,), jnp.int32) for _ in range(_NBI)],
            [pltpu.VMEM((_CHUNK,), jnp.int32) for _ in range(_NBI)],
            [pltpu.VMEM((_CHUNK, d), jnp.float32) for _ in range(_NB)],
            [pltpu.SemaphoreType.DMA for _ in range(_NBI)],
            [pltpu.SemaphoreType.DMA for _ in range(_NB)],
            [pltpu.SemaphoreType.DMA for _ in range(_NB)],
            pltpu.VMEM_SHARED((_N, d), jnp.float32),
        ],
    )
    def sc_agg(table_hbm, src_hbm, dst_hbm, et_hbm, out_hbm,
               idsb, etb, dstb, rows, sem_i, sem_g, sem_s, acc_sh):
        cid = lax.axis_index("c")
        sid = lax.axis_index("s")
        base = cid * (_E // _NC) + sid * _EW
        row0 = sid * _RPT

        zvec = jnp.zeros((16,), jnp.float32)

        @pl.loop(0, _CHUNK)
        def _zero_row(r):
            for c in range(d // 16):
                rows[0][r, pl.ds(c * 16, 16)] = zvec

        for t in range(_RPT // _CHUNK):
            pltpu.sync_copy(rows[0], acc_sh.at[pl.ds(row0 + t * _CHUNK, _CHUNK)])
        rem = _RPT % _CHUNK
        if rem:
            pltpu.sync_copy(rows[0].at[pl.ds(0, rem)],
                            acc_sh.at[pl.ds(row0 + (_RPT // _CHUNK) * _CHUNK, rem)])

        @pl.when(sid == _NS - 1)
        def _zero_tail():
            pltpu.sync_copy(rows[0].at[pl.ds(0, _TAIL)],
                            acc_sh.at[pl.ds(_TAIL0, _TAIL)])

        plsc.subcore_barrier()

        def fire_idx(j, bi):
            off = base + j * _CHUNK
            pltpu.async_copy(src_hbm.at[pl.ds(off, _CHUNK)], idsb[bi], sem_i[bi])
            pltpu.async_copy(et_hbm.at[pl.ds(off, _CHUNK)], etb[bi], sem_i[bi])
            pltpu.async_copy(dst_hbm.at[pl.ds(off, _CHUNK)], dstb[bi], sem_i[bi])

        def ready_gather(jf, bi, b4):
            pltpu.make_async_copy(src_hbm.at[pl.ds(base, _CHUNK)],
                                  idsb[bi], sem_i[bi]).wait()
            pltpu.make_async_copy(et_hbm.at[pl.ds(base, _CHUNK)],
                                  etb[bi], sem_i[bi]).wait()
            pltpu.make_async_copy(dst_hbm.at[pl.ds(base, _CHUNK)],
                                  dstb[bi], sem_i[bi]).wait()
            for t in range(_CHUNK // 16):
                sl = pl.ds(t * 16, 16)
                idsb[bi][sl] = etb[bi][sl] * _N + idsb[bi][sl]
            pltpu.async_copy(table_hbm.at[idsb[bi]], rows[b4], sem_g[b4])

        def wait_scatter(b4, b8):
            pltpu.make_async_copy(rows[b4], acc_sh.at[dstb[b8]],
                                  sem_s[b4]).wait()

        def step(b8, dyn_j, guard_ws=False, guard_fi=False, last=None):
            b4 = b8 % _NB
            pltpu.make_async_copy(table_hbm.at[idsb[b8]], rows[b4],
                                  sem_g[b4]).wait()
            pltpu.async_copy(rows[b4], acc_sh.at[dstb[b8]], sem_s[b4],
                             add=True)
            if guard_ws:
                @pl.when(dyn_j > 0)
                def _ws():
                    wait_scatter((b4 - 1) % _NB, (b8 - 1) % _NBI)
            else:
                wait_scatter((b4 - 1) % _NB, (b8 - 1) % _NBI)
            if last is None or last >= _GLEAD:
                ready_gather(dyn_j + _GLEAD, (b8 + _GLEAD) % _NBI,
                             (b4 + _GLEAD) % _NB)
            if last is None or last >= _ILEAD:
                if guard_fi:
                    @pl.when(dyn_j + _ILEAD < _NCHUNK)
                    def _fi():
                        fire_idx(dyn_j + _ILEAD, (b8 + _ILEAD) % _NBI)
                else:
                    fire_idx(dyn_j + _ILEAD, (b8 + _ILEAD) % _NBI)

        for c in range(_ILEAD):
            fire_idx(c, c)
        for c in range(_GLEAD):
            ready_gather(c, c, c)

        @pl.loop(0, _NCHUNK // _NBI)
        def _grp(g):
            j0 = g * _NBI
            for b8 in range(_NBI):
                step(b8, j0 + b8, guard_ws=(b8 == 0), guard_fi=(b8 == _NBI - 1))

        for j in range(_NCHUNK - (_NCHUNK % _NBI), _NCHUNK):
            step(j % _NBI, j, last=_NCHUNK - 1 - j)

        wait_scatter((_NCHUNK - 1) % _NB, (_NCHUNK - 1) % _NBI)

        plsc.subcore_barrier()
        pltpu.sync_copy(acc_sh.at[pl.ds(row0, _RPT)],
                        out_hbm.at[cid, pl.ds(row0, _RPT)])

        @pl.when(sid == _NS - 1)
        def _write_tail():
            pltpu.sync_copy(acc_sh.at[pl.ds(_TAIL0, _TAIL)],
                            out_hbm.at[cid, pl.ds(_TAIL0, _TAIL)])

    return sc_agg


_BN = 2000
_GRID = _N // _BN


def _mm(a, b):
    return lax.dot(a.astype(jnp.bfloat16), b.astype(jnp.bfloat16),
                   preferred_element_type=jnp.float32)


def _table0_body(h_ref, w_ref, table_ref):
    h = h_ref[...]
    w = w_ref[...]
    table_ref[0] = _mm(h, w[0])
    table_ref[1] = _mm(h, w[1])


def _tablef_body(agg_ref, selfp_ref, w_ref, table_ref, h_ref):
    h = jnp.maximum(agg_ref[0] + agg_ref[1] + selfp_ref[...], 0.0)
    w = w_ref[...]
    table_ref[0] = _mm(h, w[0])
    table_ref[1] = _mm(h, w[1])
    h_ref[...] = h


def _self_body(h_ref, wl_ref, b_ref, self_ref):
    self_ref[...] = _mm(h_ref[...], wl_ref[...]) + b_ref[...]


def _table0(h, W):
    din, dout = W.shape[1:]
    return pl.pallas_call(
        _table0_body,
        grid=(_GRID,),
        in_specs=[
            pl.BlockSpec((_BN, din), lambda i: (i, 0)),
            pl.BlockSpec((_NREL, din, dout), lambda i: (0, 0, 0)),
        ],
        out_specs=pl.BlockSpec((_NREL, _BN, dout), lambda i: (0, i, 0)),
        out_shape=jax.ShapeDtypeStruct((_NREL, _N, dout), jnp.float32),
    )(h, W)


def _tablef(agg, selfp, W):
    din, dout = W.shape[1:]
    return pl.pallas_call(
        _tablef_body,
        grid=(_GRID,),
        in_specs=[
            pl.BlockSpec((_NC, _BN, din), lambda i: (0, i, 0)),
            pl.BlockSpec((_BN, din), lambda i: (i, 0)),
            pl.BlockSpec((_NREL, din, dout), lambda i: (0, 0, 0)),
        ],
        out_specs=[
            pl.BlockSpec((_NREL, _BN, dout), lambda i: (0, i, 0)),
            pl.BlockSpec((_BN, din), lambda i: (i, 0)),
        ],
        out_shape=[
            jax.ShapeDtypeStruct((_NREL, _N, dout), jnp.float32),
            jax.ShapeDtypeStruct((_N, din), jnp.float32),
        ],
    )(agg, selfp, W)


def _selfk(h, Wl, b):
    din, dout = Wl.shape
    return pl.pallas_call(
        _self_body,
        grid=(_GRID,),
        in_specs=[
            pl.BlockSpec((_BN, din), lambda i: (i, 0)),
            pl.BlockSpec((din, dout), lambda i: (0, 0)),
            pl.BlockSpec((1, dout), lambda i: (0, 0)),
        ],
        out_specs=pl.BlockSpec((_BN, dout), lambda i: (i, 0)),
        out_shape=jax.ShapeDtypeStruct((_N, dout), jnp.float32),
    )(h, Wl, b.reshape(1, dout))


def _final_body(agg_ref, h_ref, wl_ref, b_ref, out_ref):
    out_ref[...] = (agg_ref[0] + agg_ref[1]
                    + _mm(h_ref[...], wl_ref[...]) + b_ref[...])


def _final(agg, h, Wl, b):
    din, dout = Wl.shape
    return pl.pallas_call(
        _final_body,
        grid=(_GRID,),
        in_specs=[
            pl.BlockSpec((_NC, _BN, dout), lambda i: (0, i, 0)),
            pl.BlockSpec((_BN, din), lambda i: (i, 0)),
            pl.BlockSpec((din, dout), lambda i: (0, 0)),
            pl.BlockSpec((1, dout), lambda i: (0, 0)),
        ],
        out_specs=pl.BlockSpec((_BN, dout), lambda i: (i, 0)),
        out_shape=jax.ShapeDtypeStruct((_N, dout), jnp.float32),
    )(agg, h, Wl, b.reshape(1, dout))


def kernel(features, edge_index, edge_type, W0, Wl0, b0, W1, Wl1, b1, W2, Wl2, b2):
    sc128 = _make_sc_agg(128)
    sc64 = _make_sc_agg(64)
    src = edge_index[0]
    dst = edge_index[1]
    table0 = _table0(features, W0)
    agg0 = sc128(table0.reshape(_NREL * _N, 128), src, dst, edge_type)
    self0 = _selfk(features, Wl0, b0)
    table1, h1 = _tablef(agg0, self0, W1)
    agg1 = sc128(table1.reshape(_NREL * _N, 128), src, dst, edge_type)
    self1 = _selfk(h1, Wl1, b1)
    table2, h2 = _tablef(agg1, self1, W2)
    agg2 = sc64(table2.reshape(_NREL * _N, 64), src, dst, edge_type)
    return _final(agg2, h2, Wl2, b2)


# speedup vs baseline: 1.0796x; 1.0477x over previous
import functools

import jax
import jax.numpy as jnp
from jax import lax
from jax.experimental import pallas as pl
from jax.experimental.pallas import tpu as pltpu
from jax.experimental.pallas import tpu_sc as plsc

_N = 10000
_E = 320000
_NREL = 2

_NC = 2
_NS = 16
_NW = _NC * _NS
_CHUNK = 80
_EW = _E // _NW
_NCHUNK = _EW // _CHUNK
_RPT = 624
_TAIL0 = _NS * _RPT
_TAIL = _N - _TAIL0


_NB = 4
_NBI = 8
_ILEAD = 6
_GLEAD = _NB - 1


@functools.lru_cache(maxsize=None)
def _make_sc_agg(d):
    mesh = plsc.VectorSubcoreMesh(core_axis_name="c", subcore_axis_name="s")

    @functools.partial(
        pl.kernel,
        mesh=mesh,
        compiler_params=pltpu.CompilerParams(use_tc_tiling_on_sc=False),
        out_type=jax.ShapeDtypeStruct((_NC, _N, d), jnp.bfloat16),
        scratch_types=[
            [pltpu.VMEM((_CHUNK,), jnp.int32) for _ in range(_NBI)],
            [pltpu.VMEM((_CHUNK,), jnp.int32) for _ in range(_NBI)],
            [pltpu.VMEM((_CHUNK,), jnp.int32) for _ in range(_NBI)],
            [pltpu.VMEM((_CHUNK, d), jnp.bfloat16) for _ in range(_NB)],
            [pltpu.SemaphoreType.DMA for _ in range(_NBI)],
            [pltpu.SemaphoreType.DMA for _ in range(_NB)],
            [pltpu.SemaphoreType.DMA for _ in range(_NB)],
            pltpu.VMEM_SHARED((_N, d), jnp.bfloat16),
        ],
    )
    def sc_agg(table_hbm, src_hbm, dst_hbm, et_hbm, out_hbm,
               idsb, etb, dstb, rows, sem_i, sem_g, sem_s, acc_sh):
        cid = lax.axis_index("c")
        sid = lax.axis_index("s")
        base = cid * (_E // _NC) + sid * _EW
        row0 = sid * _RPT

        zvec = jnp.zeros((32,), jnp.bfloat16)

        @pl.loop(0, _CHUNK)
        def _zero_row(r):
            for c in range(d // 32):
                rows[0][r, pl.ds(c * 32, 32)] = zvec

        for t in range(_RPT // _CHUNK):
            pltpu.sync_copy(rows[0], acc_sh.at[pl.ds(row0 + t * _CHUNK, _CHUNK)])
        rem = _RPT % _CHUNK
        if rem:
            pltpu.sync_copy(rows[0].at[pl.ds(0, rem)],
                            acc_sh.at[pl.ds(row0 + (_RPT // _CHUNK) * _CHUNK, rem)])

        @pl.when(sid == _NS - 1)
        def _zero_tail():
            pltpu.sync_copy(rows[0].at[pl.ds(0, _TAIL)],
                            acc_sh.at[pl.ds(_TAIL0, _TAIL)])

        plsc.subcore_barrier()

        def fire_idx(j, bi):
            off = base + j * _CHUNK
            pltpu.async_copy(src_hbm.at[pl.ds(off, _CHUNK)], idsb[bi], sem_i[bi])
            pltpu.async_copy(et_hbm.at[pl.ds(off, _CHUNK)], etb[bi], sem_i[bi])
            pltpu.async_copy(dst_hbm.at[pl.ds(off, _CHUNK)], dstb[bi], sem_i[bi])

        def ready_gather(jf, bi, b4):
            pltpu.make_async_copy(src_hbm.at[pl.ds(base, _CHUNK)],
                                  idsb[bi], sem_i[bi]).wait()
            pltpu.make_async_copy(et_hbm.at[pl.ds(base, _CHUNK)],
                                  etb[bi], sem_i[bi]).wait()
            pltpu.make_async_copy(dst_hbm.at[pl.ds(base, _CHUNK)],
                                  dstb[bi], sem_i[bi]).wait()
            for t in range(_CHUNK // 16):
                sl = pl.ds(t * 16, 16)
                idsb[bi][sl] = etb[bi][sl] * _N + idsb[bi][sl]
            pltpu.async_copy(table_hbm.at[idsb[bi]], rows[b4], sem_g[b4])

        def wait_scatter(b4, b8):
            pltpu.make_async_copy(rows[b4], acc_sh.at[dstb[b8]],
                                  sem_s[b4]).wait()

        def step(b8, dyn_j, guard_ws=False, guard_fi=False, last=None):
            b4 = b8 % _NB
            pltpu.make_async_copy(table_hbm.at[idsb[b8]], rows[b4],
                                  sem_g[b4]).wait()
            pltpu.async_copy(rows[b4], acc_sh.at[dstb[b8]], sem_s[b4],
                             add=True)
            if guard_ws:
                @pl.when(dyn_j > 0)
                def _ws():
                    wait_scatter((b4 - 1) % _NB, (b8 - 1) % _NBI)
            else:
                wait_scatter((b4 - 1) % _NB, (b8 - 1) % _NBI)
            if last is None or last >= _GLEAD:
                ready_gather(dyn_j + _GLEAD, (b8 + _GLEAD) % _NBI,
                             (b4 + _GLEAD) % _NB)
            if last is None or last >= _ILEAD:
                if guard_fi:
                    @pl.when(dyn_j + _ILEAD < _NCHUNK)
                    def _fi():
                        fire_idx(dyn_j + _ILEAD, (b8 + _ILEAD) % _NBI)
                else:
                    fire_idx(dyn_j + _ILEAD, (b8 + _ILEAD) % _NBI)

        for c in range(_ILEAD):
            fire_idx(c, c)
        for c in range(_GLEAD):
            ready_gather(c, c, c)

        @pl.loop(0, _NCHUNK // _NBI)
        def _grp(g):
            j0 = g * _NBI
            for b8 in range(_NBI):
                step(b8, j0 + b8, guard_ws=(b8 == 0), guard_fi=(b8 == _NBI - 1))

        for j in range(_NCHUNK - (_NCHUNK % _NBI), _NCHUNK):
            step(j % _NBI, j, last=_NCHUNK - 1 - j)

        wait_scatter((_NCHUNK - 1) % _NB, (_NCHUNK - 1) % _NBI)

        plsc.subcore_barrier()
        pltpu.sync_copy(acc_sh.at[pl.ds(row0, _RPT)],
                        out_hbm.at[cid, pl.ds(row0, _RPT)])

        @pl.when(sid == _NS - 1)
        def _write_tail():
            pltpu.sync_copy(acc_sh.at[pl.ds(_TAIL0, _TAIL)],
                            out_hbm.at[cid, pl.ds(_TAIL0, _TAIL)])

    return sc_agg


_BN = 2000
_GRID = _N // _BN


def _mm(a, b):
    return lax.dot(a.astype(jnp.bfloat16), b.astype(jnp.bfloat16),
                   preferred_element_type=jnp.float32)


def _table0_body(h_ref, w_ref, table_ref):
    h = h_ref[...]
    w = w_ref[...]
    table_ref[0] = _mm(h, w[0]).astype(jnp.bfloat16)
    table_ref[1] = _mm(h, w[1]).astype(jnp.bfloat16)


def _tablef_body(agg_ref, selfp_ref, w_ref, table_ref, h_ref):
    h = jnp.maximum(agg_ref[0].astype(jnp.float32)
                    + agg_ref[1].astype(jnp.float32) + selfp_ref[...], 0.0)
    w = w_ref[...]
    table_ref[0] = _mm(h, w[0]).astype(jnp.bfloat16)
    table_ref[1] = _mm(h, w[1]).astype(jnp.bfloat16)
    h_ref[...] = h


def _self_body(h_ref, wl_ref, b_ref, self_ref):
    self_ref[...] = _mm(h_ref[...], wl_ref[...]) + b_ref[...]


def _table0(h, W):
    din, dout = W.shape[1:]
    return pl.pallas_call(
        _table0_body,
        grid=(_GRID,),
        in_specs=[
            pl.BlockSpec((_BN, din), lambda i: (i, 0)),
            pl.BlockSpec((_NREL, din, dout), lambda i: (0, 0, 0)),
        ],
        out_specs=pl.BlockSpec((_NREL, _BN, dout), lambda i: (0, i, 0)),
        out_shape=jax.ShapeDtypeStruct((_NREL, _N, dout), jnp.bfloat16),
    )(h, W)


def _tablef(agg, selfp, W):
    din, dout = W.shape[1:]
    return pl.pallas_call(
        _tablef_body,
        grid=(_GRID,),
        in_specs=[
            pl.BlockSpec((_NC, _BN, din), lambda i: (0, i, 0)),
            pl.BlockSpec((_BN, din), lambda i: (i, 0)),
            pl.BlockSpec((_NREL, din, dout), lambda i: (0, 0, 0)),
        ],
        out_specs=[
            pl.BlockSpec((_NREL, _BN, dout), lambda i: (0, i, 0)),
            pl.BlockSpec((_BN, din), lambda i: (i, 0)),
        ],
        out_shape=[
            jax.ShapeDtypeStruct((_NREL, _N, dout), jnp.bfloat16),
            jax.ShapeDtypeStruct((_N, din), jnp.float32),
        ],
    )(agg, selfp, W)


def _selfk(h, Wl, b):
    din, dout = Wl.shape
    return pl.pallas_call(
        _self_body,
        grid=(_GRID,),
        in_specs=[
            pl.BlockSpec((_BN, din), lambda i: (i, 0)),
            pl.BlockSpec((din, dout), lambda i: (0, 0)),
            pl.BlockSpec((1, dout), lambda i: (0, 0)),
        ],
        out_specs=pl.BlockSpec((_BN, dout), lambda i: (i, 0)),
        out_shape=jax.ShapeDtypeStruct((_N, dout), jnp.float32),
    )(h, Wl, b.reshape(1, dout))


def _final_body(agg_ref, h_ref, wl_ref, b_ref, out_ref):
    out_ref[...] = (agg_ref[0].astype(jnp.float32)
                    + agg_ref[1].astype(jnp.float32)
                    + _mm(h_ref[...], wl_ref[...]) + b_ref[...])


def _final(agg, h, Wl, b):
    din, dout = Wl.shape
    return pl.pallas_call(
        _final_body,
        grid=(_GRID,),
        in_specs=[
            pl.BlockSpec((_NC, _BN, dout), lambda i: (0, i, 0)),
            pl.BlockSpec((_BN, din), lambda i: (i, 0)),
            pl.BlockSpec((din, dout), lambda i: (0, 0)),
            pl.BlockSpec((1, dout), lambda i: (0, 0)),
        ],
        out_specs=pl.BlockSpec((_BN, dout), lambda i: (i, 0)),
        out_shape=jax.ShapeDtypeStruct((_N, dout), jnp.float32),
    )(agg, h, Wl, b.reshape(1, dout))


def kernel(features, edge_index, edge_type, W0, Wl0, b0, W1, Wl1, b1, W2, Wl2, b2):
    sc128 = _make_sc_agg(128)
    sc64 = _make_sc_agg(64)
    src = edge_index[0]
    dst = edge_index[1]
    table0 = _table0(features, W0)
    agg0 = sc128(table0.reshape(_NREL * _N, 128), src, dst, edge_type)
    self0 = _selfk(features, Wl0, b0)
    table1, h1 = _tablef(agg0, self0, W1)
    agg1 = sc128(table1.reshape(_NREL * _N, 128), src, dst, edge_type)
    self1 = _selfk(h1, Wl1, b1)
    table2, h2 = _tablef(agg1, self1, W2)
    agg2 = sc64(table2.reshape(_NREL * _N, 64), src, dst, edge_type)
    return _final(agg2, h2, Wl2, b2)
